# baseline (speedup 1.0000x reference)
import functools

import jax
import jax.numpy as jnp
from jax import lax
from jax.experimental import pallas as pl
from jax.experimental.pallas import tpu as pltpu
from jax.experimental.pallas import tpu_sc as plsc

_STRIDES = (8.0, 16.0, 32.0, 64.0, 128.0)
_SIZES = (16384, 4096, 1024, 256, 64)
_P = sum(_SIZES)
_G = 500
_RPW = 704
_NW = 31
_TROW = 88
_NTILE = _RPW // _TROW
_MW = 125


def _sc_body(xs_hbm, ys_hbm, ss_hbm, gtx0_hbm, gty0_hbm, gtx1_hbm, gty1_hbm,
             lab_hbm, mnx_hbm, mny_hbm, sgx_hbm, sgy_hbm, mask_hbm, out_hbm,
             uv, vv, sv, cxp, cyp, mxp, myp, axp, ayp,
             x0v, y0v, x1v, y1v, labv, mnxv, mnyv, sgxv, sgyv,
             maskv, outv):
    wid = lax.axis_index("s") * 2 + lax.axis_index("c")

    @pl.when(wid < _NW)
    def _work():
        iota = lax.iota(jnp.int32, 16)
        row0 = wid * _RPW

        pltpu.sync_copy(xs_hbm.at[pl.ds(row0, _RPW)], uv)
        pltpu.sync_copy(ys_hbm.at[pl.ds(row0, _RPW)], vv)
        pltpu.sync_copy(ss_hbm.at[pl.ds(row0, _RPW)], sv)
        pltpu.sync_copy(gtx0_hbm, x0v)
        pltpu.sync_copy(gty0_hbm, y0v)
        pltpu.sync_copy(gtx1_hbm, x1v)
        pltpu.sync_copy(gty1_hbm, y1v)
        pltpu.sync_copy(lab_hbm, labv)
        pltpu.sync_copy(mnx_hbm, mnxv)
        pltpu.sync_copy(mny_hbm, mnyv)
        pltpu.sync_copy(sgx_hbm, sgxv)
        pltpu.sync_copy(sgy_hbm, sgyv)

        for c in range(_RPW // 16):
            sl = pl.ds(16 * c, 16)
            s16 = sv[sl]
            uv[sl] = uv[sl] * s16
            vv[sl] = vv[sl] * s16

        pidx = (iota & 3) * 128 + (iota >> 2)
        for c in range(32):
            sl = pl.ds(16 * c, 16)
            idx = pidx + 4 * c
            cx16 = (x0v[sl] + x1v[sl]) * 0.5
            cy16 = (y0v[sl] + y1v[sl]) * 0.5
            lab16 = labv[sl]
            mx16 = plsc.load_gather(mnxv, [lab16])
            my16 = plsc.load_gather(mnyv, [lab16])
            sx16 = plsc.load_gather(sgxv, [lab16])
            sy16 = plsc.load_gather(sgyv, [lab16])
            plsc.store_scatter(cxp, [idx], cx16)
            plsc.store_scatter(cyp, [idx], cy16)
            plsc.store_scatter(mxp, [idx], mx16)
            plsc.store_scatter(myp, [idx], my16)
            plsc.store_scatter(axp, [idx], -0.5 / (sx16 * sx16))
            plsc.store_scatter(ayp, [idx], -0.5 / (sy16 * sy16))

        mask_w0 = row0 * _MW
        out_w0 = row0 * _G
        four_iota = 4 * iota
        tail_mask = iota < 13

        def _tile(t, _):
            pltpu.sync_copy(
                mask_hbm.at[pl.ds(mask_w0 + t * (_TROW * _MW), _TROW * _MW)],
                maskv.at[pl.ds(0, _TROW * _MW)])
            for k in range(4):
                def _gchunk(c, _, k=k):
                    psl = pl.ds(k * 128 + 16 * c, 16)
                    cx16 = cxp[psl]
                    cy16 = cyp[psl]
                    mx16 = mxp[psl]
                    my16 = myp[psl]
                    nax16 = axp[psl]
                    nay16 = ayp[psl]
                    col0 = four_iota + (64 * c + k)
                    cvec = jnp.zeros((16,), jnp.int32) + c
                    vmask = (cvec < 7) | tail_mask

                    def _row(p, _):
                        pg = t * _TROW + p
                        sp = jnp.zeros((16,), jnp.int32) + pg
                        u16 = plsc.load_gather(uv, [sp])
                        v16 = plsc.load_gather(vv, [sp])
                        s16 = plsc.load_gather(sv, [sp])
                        w16 = maskv[pl.ds(p * _MW + 16 * c, 16)]
                        mb = (w16 >> (8 * k)) & 0xFF
                        d1 = u16 - cx16 * s16 - mx16
                        d2 = v16 - cy16 * s16 - my16
                        tt = nax16 * (d1 * d1) + nay16 * (d2 * d2)
                        wgt = jnp.where(mb != 0, jnp.exp(tt), 0.0)
                        colv = col0 + p * _G
                        plsc.store_scatter(outv, [colv], wgt, mask=vmask)
                        return _

                    lax.fori_loop(0, _TROW, _row, 0, unroll=8)
                    return _

                lax.fori_loop(0, 8, _gchunk, 0, unroll=False)
            pltpu.sync_copy(
                outv,
                out_hbm.at[pl.ds(out_w0 + t * (_TROW * _G), _TROW * _G)])
            return _

        lax.fori_loop(0, _NTILE, _tile, 0, unroll=False)


@jax.jit
def _center_prior_sc(xs, ys, ss, gtx0, gty0, gtx1, gty1,
                     lab, mnx, mny, sgx, sgy, mask_words):
    mesh = plsc.VectorSubcoreMesh(core_axis_name="c", subcore_axis_name="s")
    f = functools.partial(
        pl.kernel,
        mesh=mesh,
        out_type=jax.ShapeDtypeStruct((_P * _G,), jnp.float32),
        scratch_types=[
            pltpu.VMEM((_RPW,), jnp.float32),
            pltpu.VMEM((_RPW,), jnp.float32),
            pltpu.VMEM((_RPW,), jnp.float32),
            pltpu.VMEM((512,), jnp.float32),
            pltpu.VMEM((512,), jnp.float32),
            pltpu.VMEM((512,), jnp.float32),
            pltpu.VMEM((512,), jnp.float32),
            pltpu.VMEM((512,), jnp.float32),
            pltpu.VMEM((512,), jnp.float32),
            pltpu.VMEM((512,), jnp.float32),
            pltpu.VMEM((512,), jnp.float32),
            pltpu.VMEM((512,), jnp.float32),
            pltpu.VMEM((512,), jnp.float32),
            pltpu.VMEM((512,), jnp.int32),
            pltpu.VMEM((128,), jnp.float32),
            pltpu.VMEM((128,), jnp.float32),
            pltpu.VMEM((128,), jnp.float32),
            pltpu.VMEM((128,), jnp.float32),
            pltpu.VMEM((_TROW * _MW + 16,), jnp.int32),
            pltpu.VMEM((_TROW * _G,), jnp.float32),
        ],
        compiler_params=pltpu.CompilerParams(needs_layout_passes=False),
    )(_sc_body)
    return f(xs, ys, ss, gtx0, gty0, gtx1, gty1, lab, mnx, mny, sgx, sgy,
             mask_words)


def kernel(points0, points1, points2, points3, points4,
           gt_bboxes, labels, inside_gt_bbox_mask, mean, sigma):
    pts = jnp.concatenate([points0, points1, points2, points3, points4], axis=0)
    inv_s = jnp.repeat(
        jnp.asarray([1.0 / s for s in _STRIDES], jnp.float32),
        jnp.asarray(_SIZES),
        total_repeat_length=_P,
    )
    xs = pts[:, 0]
    ys = pts[:, 1]

    gtx0 = jnp.zeros((512,), jnp.float32).at[:_G].set(gt_bboxes[:, 0])
    gty0 = jnp.zeros((512,), jnp.float32).at[:_G].set(gt_bboxes[:, 1])
    gtx1 = jnp.zeros((512,), jnp.float32).at[:_G].set(gt_bboxes[:, 2])
    gty1 = jnp.zeros((512,), jnp.float32).at[:_G].set(gt_bboxes[:, 3])
    lab = jnp.zeros((512,), jnp.int32).at[:_G].set(labels.astype(jnp.int32))
    mnx = jnp.zeros((128,), jnp.float32).at[:80].set(mean[:, 0])
    mny = jnp.zeros((128,), jnp.float32).at[:80].set(mean[:, 1])
    sgx = jnp.ones((128,), jnp.float32).at[:80].set(sigma[:, 0])
    sgy = jnp.ones((128,), jnp.float32).at[:80].set(sigma[:, 1])
    mask_words = inside_gt_bbox_mask.reshape(-1).view(jnp.int32)

    w = _center_prior_sc(xs, ys, inv_s, gtx0, gty0, gtx1, gty1,
                         lab, mnx, mny, sgx, sgy, mask_words)
    return (w.reshape(_P, _G), inside_gt_bbox_mask)


# speedup vs baseline: 1.3265x; 1.3265x over previous
import functools

import jax
import jax.numpy as jnp
from jax import lax
from jax.experimental import pallas as pl
from jax.experimental.pallas import tpu as pltpu
from jax.experimental.pallas import tpu_sc as plsc

_STRIDES = (8.0, 16.0, 32.0, 64.0, 128.0)
_SIZES = (16384, 4096, 1024, 256, 64)
_P = sum(_SIZES)
_G = 500
_G_PAD = 512
_ROW_BLK = 1024
_K = 16


def _body(pts_ref, gt_ref, lab_ref, mean_ref, sig_ref, mask_ref, out_ref, b_ref):
    i = pl.program_id(0)

    @pl.when(i == 0)
    def _init():
        cx = (gt_ref[0:1, :] + gt_ref[2:3, :]) * 0.5
        cy = (gt_ref[1:2, :] + gt_ref[3:4, :]) * 0.5
        lab = lab_ref[0:1, :]
        cls = jax.lax.broadcasted_iota(jnp.int32, (128, _G_PAD), 0)
        oh = (jnp.broadcast_to(lab, (128, _G_PAD)) == cls).astype(jnp.float32)
        mx = jnp.sum(oh * mean_ref[:, 0:1], axis=0, keepdims=True)
        my = jnp.sum(oh * mean_ref[:, 1:2], axis=0, keepdims=True)
        sx = jnp.sum(oh * sig_ref[:, 0:1], axis=0, keepdims=True)
        sy = jnp.sum(oh * sig_ref[:, 1:2], axis=0, keepdims=True)
        ax = 0.5 / (sx * sx)
        ay = 0.5 / (sy * sy)
        rows = (
            -ax,
            2.0 * ax * cx,
            2.0 * ax * mx,
            -ay,
            2.0 * ay * cy,
            2.0 * ay * my,
            -(ax * cx * cx + ay * cy * cy),
            -2.0 * (ax * cx * mx + ay * cy * my),
            -(ax * mx * mx + ay * my * my),
        )
        for k, r in enumerate(rows):
            b_ref[k : k + 1, :] = r
        b_ref[9:16, :] = jnp.zeros((7, _G_PAD), jnp.float32)

    x = pts_ref[:, 0:1]
    y = pts_ref[:, 1:2]
    s = pts_ref[:, 2:3]
    u = x * s
    v = y * s
    cols = (u * u, u * s, u, v * v, v * s, v, s * s, s, jnp.ones_like(s))
    lane = jax.lax.broadcasted_iota(jnp.int32, (_ROW_BLK, _K), 1)
    a = jnp.zeros((_ROW_BLK, _K), jnp.float32)
    for k, c in enumerate(cols):
        a = jnp.where(lane == k, jnp.broadcast_to(c, (_ROW_BLK, _K)), a)
    t = jax.lax.dot_general(
        a,
        b_ref[...],
        dimension_numbers=(((1,), (0,)), ((), ())),
        preferred_element_type=jnp.float32,
        precision=jax.lax.Precision.HIGHEST,
    )
    w = jnp.exp(t)
    out_ref[...] = jnp.where(mask_ref[...], w, 0.0)


@functools.partial(jax.jit, static_argnames=())
def _center_prior_tc(pts3, gt_t, lab_p, mean_p, sig_p, mask):
    grid = (pl.cdiv(_P, _ROW_BLK),)
    return pl.pallas_call(
        _body,
        grid=grid,
        in_specs=[
            pl.BlockSpec((_ROW_BLK, 4), lambda i: (i, 0)),
            pl.BlockSpec((8, _G_PAD), lambda i: (0, 0)),
            pl.BlockSpec((8, _G_PAD), lambda i: (0, 0)),
            pl.BlockSpec((128, 128), lambda i: (0, 0)),
            pl.BlockSpec((128, 128), lambda i: (0, 0)),
            pl.BlockSpec((_ROW_BLK, _G_PAD), lambda i: (i, 0)),
        ],
        out_specs=pl.BlockSpec((_ROW_BLK, _G_PAD), lambda i: (i, 0)),
        out_shape=jax.ShapeDtypeStruct((_P, _G), jnp.float32),
        scratch_shapes=[pltpu.VMEM((_K, _G_PAD), jnp.float32)],
        compiler_params=pltpu.CompilerParams(
            dimension_semantics=("arbitrary",),
        ),
    )(pts3, gt_t, lab_p, mean_p, sig_p, mask)


_MASK_WORDS = _P * _G // 4
_WPW = 85248
_WCH = _WPW // 2


def _sc_copy_body(src_hbm, dst_hbm, buf):
    wid = lax.axis_index("s") * 2 + lax.axis_index("c")
    base = wid * _WPW

    def _chunk(c, _):
        off = base + c * _WCH
        pltpu.sync_copy(src_hbm.at[pl.ds(off, _WCH)], buf.at[pl.ds(0, _WCH)])
        pltpu.sync_copy(buf.at[pl.ds(0, _WCH)], dst_hbm.at[pl.ds(off, _WCH)])
        return _

    lax.fori_loop(0, 2, _chunk, 0, unroll=False)

    @pl.when(wid == 0)
    def _tail():
        t0 = 32 * _WPW
        pltpu.sync_copy(src_hbm.at[pl.ds(t0, 64)], buf.at[pl.ds(0, 64)])
        pltpu.sync_copy(buf.at[pl.ds(0, 64)], dst_hbm.at[pl.ds(t0, 64)])


@jax.jit
def _mask_copy_sc(mask_words):
    mesh = plsc.VectorSubcoreMesh(core_axis_name="c", subcore_axis_name="s")
    f = functools.partial(
        pl.kernel,
        mesh=mesh,
        out_type=jax.ShapeDtypeStruct((_MASK_WORDS,), jnp.int32),
        scratch_types=[pltpu.VMEM((_WCH,), jnp.int32)],
        compiler_params=pltpu.CompilerParams(needs_layout_passes=False),
    )(_sc_copy_body)
    return f(mask_words)


def kernel(points0, points1, points2, points3, points4,
           gt_bboxes, labels, inside_gt_bbox_mask, mean, sigma):
    pts = jnp.concatenate([points0, points1, points2, points3, points4], axis=0)
    inv_s = jnp.repeat(
        jnp.asarray([1.0 / s for s in _STRIDES], jnp.float32),
        jnp.asarray(_SIZES),
        total_repeat_length=_P,
    )
    pts3 = jnp.concatenate(
        [pts, inv_s[:, None], jnp.zeros((_P, 1), jnp.float32)], axis=1)

    gt_t = jnp.zeros((8, _G_PAD), jnp.float32).at[:4, :_G].set(gt_bboxes.T)
    lab_p = jnp.zeros((8, _G_PAD), jnp.int32).at[0, :_G].set(labels.astype(jnp.int32))
    mean_p = jnp.zeros((128, 128), jnp.float32).at[:80, :2].set(mean)
    sig_p = jnp.ones((128, 128), jnp.float32).at[:80, :2].set(sigma)

    mask_words = inside_gt_bbox_mask.reshape(-1).view(jnp.int32)
    m_out = _mask_copy_sc(mask_words)
    m_out = m_out.view(jnp.int8).view(jnp.bool_).reshape(_P, _G)
    w = _center_prior_tc(pts3, gt_t, lab_p, mean_p, sig_p, inside_gt_bbox_mask)
    return (w, m_out)


# speedup vs baseline: 10.9815x; 8.2788x over previous
import functools

import jax
import jax.numpy as jnp
from jax.experimental import pallas as pl
from jax.experimental.pallas import tpu as pltpu

_STRIDES = (8.0, 16.0, 32.0, 64.0, 128.0)
_SIZES = (16384, 4096, 1024, 256, 64)
_P = sum(_SIZES)
_G = 500
_G_PAD = 512
_ROW_BLK = 1024
_K = 16


def _body(pts_ref, gt_ref, lab_ref, mean_ref, sig_ref, mask_ref, out_ref, b_ref):
    i = pl.program_id(0)

    @pl.when(i == 0)
    def _init():
        cx = (gt_ref[0:1, :] + gt_ref[2:3, :]) * 0.5
        cy = (gt_ref[1:2, :] + gt_ref[3:4, :]) * 0.5
        lab = lab_ref[0:1, :]
        cls = jax.lax.broadcasted_iota(jnp.int32, (128, _G_PAD), 0)
        oh = (jnp.broadcast_to(lab, (128, _G_PAD)) == cls).astype(jnp.float32)
        mx = jnp.sum(oh * mean_ref[:, 0:1], axis=0, keepdims=True)
        my = jnp.sum(oh * mean_ref[:, 1:2], axis=0, keepdims=True)
        sx = jnp.sum(oh * sig_ref[:, 0:1], axis=0, keepdims=True)
        sy = jnp.sum(oh * sig_ref[:, 1:2], axis=0, keepdims=True)
        ax = 0.5 / (sx * sx)
        ay = 0.5 / (sy * sy)
        rows = (
            -ax,
            2.0 * ax * cx,
            2.0 * ax * mx,
            -ay,
            2.0 * ay * cy,
            2.0 * ay * my,
            -(ax * cx * cx + ay * cy * cy),
            -2.0 * (ax * cx * mx + ay * cy * my),
            -(ax * mx * mx + ay * my * my),
        )
        for k, r in enumerate(rows):
            b_ref[k : k + 1, :] = r
        b_ref[9:16, :] = jnp.zeros((7, _G_PAD), jnp.float32)

    x = pts_ref[:, 0:1]
    y = pts_ref[:, 1:2]
    s = pts_ref[:, 2:3]
    u = x * s
    v = y * s
    cols = (u * u, u * s, u, v * v, v * s, v, s * s, s, jnp.ones_like(s))
    lane = jax.lax.broadcasted_iota(jnp.int32, (_ROW_BLK, _K), 1)
    a = jnp.zeros((_ROW_BLK, _K), jnp.float32)
    for k, c in enumerate(cols):
        a = jnp.where(lane == k, jnp.broadcast_to(c, (_ROW_BLK, _K)), a)
    t = jax.lax.dot_general(
        a,
        b_ref[...],
        dimension_numbers=(((1,), (0,)), ((), ())),
        preferred_element_type=jnp.float32,
        precision=jax.lax.Precision.HIGHEST,
    )
    w = jnp.exp(t)
    out_ref[...] = jnp.where(mask_ref[...], w, 0.0)


@functools.partial(jax.jit, static_argnames=())
def _center_prior_tc(pts3, gt_t, lab_p, mean_p, sig_p, mask):
    grid = (pl.cdiv(_P, _ROW_BLK),)
    return pl.pallas_call(
        _body,
        grid=grid,
        in_specs=[
            pl.BlockSpec((_ROW_BLK, 4), lambda i: (i, 0)),
            pl.BlockSpec((8, _G_PAD), lambda i: (0, 0)),
            pl.BlockSpec((8, _G_PAD), lambda i: (0, 0)),
            pl.BlockSpec((128, 128), lambda i: (0, 0)),
            pl.BlockSpec((128, 128), lambda i: (0, 0)),
            pl.BlockSpec((_ROW_BLK, _G_PAD), lambda i: (i, 0)),
        ],
        out_specs=pl.BlockSpec((_ROW_BLK, _G_PAD), lambda i: (i, 0)),
        out_shape=jax.ShapeDtypeStruct((_P, _G), jnp.float32),
        scratch_shapes=[pltpu.VMEM((_K, _G_PAD), jnp.float32)],
        compiler_params=pltpu.CompilerParams(
            dimension_semantics=("arbitrary",),
        ),
    )(pts3, gt_t, lab_p, mean_p, sig_p, mask)


def kernel(points0, points1, points2, points3, points4,
           gt_bboxes, labels, inside_gt_bbox_mask, mean, sigma):
    pts = jnp.concatenate([points0, points1, points2, points3, points4], axis=0)
    inv_s = jnp.repeat(
        jnp.asarray([1.0 / s for s in _STRIDES], jnp.float32),
        jnp.asarray(_SIZES),
        total_repeat_length=_P,
    )
    pts3 = jnp.concatenate(
        [pts, inv_s[:, None], jnp.zeros((_P, 1), jnp.float32)], axis=1)

    gt_t = jnp.zeros((8, _G_PAD), jnp.float32).at[:4, :_G].set(gt_bboxes.T)
    lab_p = jnp.zeros((8, _G_PAD), jnp.int32).at[0, :_G].set(labels.astype(jnp.int32))
    mean_p = jnp.zeros((128, 128), jnp.float32).at[:80, :2].set(mean)
    sig_p = jnp.ones((128, 128), jnp.float32).at[:80, :2].set(sigma)

    w = _center_prior_tc(pts3, gt_t, lab_p, mean_p, sig_p, inside_gt_bbox_mask)
    return (w, inside_gt_bbox_mask)
